# Initial kernel scaffold; baseline (speedup 1.0000x reference)
import jax
import jax.numpy as jnp
from jax.experimental import pallas as pl


def kernel(x, table):
    raise NotImplementedError("write your pallas kernel here")


# speedup vs baseline: 1.7479x; 1.7479x over previous
import functools

import jax
import jax.numpy as jnp
from jax import lax
from jax.experimental import pallas as pl
from jax.experimental.pallas import tpu as pltpu
from jax.experimental.pallas import tpu_sc as plsc

NC = 2
NS = 16
NW = NC * NS
GROUP = 128


def _body(ngrp, x_hbm, table_hbm, out_hbm, idx_v, rows_v, sem0, sem1):
  wid = lax.axis_index("s") * NC + lax.axis_index("c")
  pltpu.sync_copy(x_hbm.at[wid], idx_v)

  def gather_start(g, buf, sem):
    pltpu.async_copy(table_hbm.at[idx_v.at[g]], rows_v.at[buf], sem)

  def gather_wait(g, buf, sem):
    pltpu.make_async_copy(table_hbm.at[idx_v.at[g]], rows_v.at[buf], sem).wait()

  gather_start(0, 0, sem0)

  def pair(i, carry):
    g0 = 2 * i
    g1 = g0 + 1
    gather_wait(g0, 0, sem0)
    gather_start(g1, 1, sem1)
    pltpu.sync_copy(rows_v.at[0], out_hbm.at[wid, g0])
    gather_wait(g1, 1, sem1)

    @pl.when(g1 + 1 < ngrp)
    def _():
      gather_start(g1 + 1, 0, sem0)

    pltpu.sync_copy(rows_v.at[1], out_hbm.at[wid, g1])
    return carry

  lax.fori_loop(0, ngrp // 2, pair, 0)


@jax.jit
def kernel(x, table):
  b, l = x.shape
  emb = table.shape[1]
  tot = b * l
  assert tot % (NW * GROUP) == 0
  ngrp = tot // (NW * GROUP)
  xf = x.reshape(NW, ngrp, GROUP).astype(jnp.int32)

  mesh = plsc.VectorSubcoreMesh(core_axis_name="c", subcore_axis_name="s")
  k = pl.kernel(
      functools.partial(_body, ngrp),
      out_type=jax.ShapeDtypeStruct((NW, ngrp, GROUP, emb), jnp.float32),
      mesh=mesh,
      scratch_types=[
          pltpu.VMEM((ngrp, GROUP), jnp.int32),
          pltpu.VMEM((2, GROUP, emb), jnp.float32),
          pltpu.SemaphoreType.DMA,
          pltpu.SemaphoreType.DMA,
      ],
      compiler_params=pltpu.CompilerParams(use_tc_tiling_on_sc=False),
  )
  out = k(xf, table)
  return out.reshape(b, l, emb)


# speedup vs baseline: 1.8729x; 1.0715x over previous
import functools

import jax
import jax.numpy as jnp
from jax import lax
from jax.experimental import pallas as pl
from jax.experimental.pallas import tpu as pltpu
from jax.experimental.pallas import tpu_sc as plsc

NC = 2
NS = 16
NW = NC * NS
GROUP = 128
NBUF = 10


def _body(ngrp, x_hbm, table_hbm, out_hbm, idx_v, rows_v, *sems):
  gsems = sems[:NBUF]
  osems = sems[NBUF:]
  wid = lax.axis_index("s") * NC + lax.axis_index("c")
  pltpu.sync_copy(x_hbm.at[wid], idx_v)

  def gather_start(g, b):
    pltpu.async_copy(table_hbm.at[idx_v.at[g]], rows_v.at[b], gsems[b])

  def gather_wait(g, b):
    pltpu.make_async_copy(table_hbm.at[idx_v.at[g]], rows_v.at[b],
                          gsems[b]).wait()

  def out_start(g, b):
    pltpu.async_copy(rows_v.at[b], out_hbm.at[wid, g], osems[b])

  def out_wait(g, b):
    pltpu.make_async_copy(rows_v.at[b], out_hbm.at[wid, g], osems[b]).wait()

  for b in range(NBUF):
    gather_start(b, b)

  def step(i, carry):
    base = i * NBUF
    for b in range(NBUF):
      gather_wait(base + b, b)
      out_start(base + b, b)
    for b in range(NBUF):
      out_wait(base + b, b)

      @pl.when(base + b + NBUF < ngrp)
      def _(b=b):
        gather_start(base + b + NBUF, b)

    return carry

  lax.fori_loop(0, ngrp // NBUF, step, 0)


@jax.jit
def kernel(x, table):
  b, l = x.shape
  emb = table.shape[1]
  tot = b * l
  assert tot % (NW * GROUP) == 0
  ngrp = tot // (NW * GROUP)
  assert ngrp % NBUF == 0
  xf = x.reshape(NW, ngrp, GROUP).astype(jnp.int32)

  mesh = plsc.VectorSubcoreMesh(core_axis_name="c", subcore_axis_name="s")
  k = pl.kernel(
      functools.partial(_body, ngrp),
      out_type=jax.ShapeDtypeStruct((NW, ngrp, GROUP, emb), jnp.float32),
      mesh=mesh,
      scratch_types=[
          pltpu.VMEM((ngrp, GROUP), jnp.int32),
          pltpu.VMEM((NBUF, GROUP, emb), jnp.float32),
      ] + [pltpu.SemaphoreType.DMA] * (2 * NBUF),
      compiler_params=pltpu.CompilerParams(use_tc_tiling_on_sc=False),
  )
  out = k(xf, table)
  return out.reshape(b, l, emb)
